# baseline (speedup 1.0000x reference)
import functools

import jax
import jax.numpy as jnp
from jax.experimental import pallas as pl
from jax.experimental.pallas import tpu as pltpu

NUM_EXPERTS = 8
TOP_K = 2
HIDDEN = 768
INTER = 2048
T_TOKENS = 2048

TB = 256
IB = 512


def _router_body(x_ref, gw_ref, comb_ref):
    x = x_ref[...]
    logits = jnp.dot(x, gw_ref[...], preferred_element_type=jnp.float32)
    m = jnp.max(logits, axis=1, keepdims=True)
    ex = jnp.exp(logits - m)
    p = ex / jnp.sum(ex, axis=1, keepdims=True)
    iota = jax.lax.broadcasted_iota(jnp.int32, (p.shape[0], NUM_EXPERTS), 1)
    m1 = jnp.max(p, axis=1, keepdims=True)
    i1 = jnp.min(jnp.where(p == m1, iota, NUM_EXPERTS), axis=1, keepdims=True)
    is1 = iota == i1
    p2 = jnp.where(is1, -1.0, p)
    m2 = jnp.max(p2, axis=1, keepdims=True)
    i2 = jnp.min(jnp.where(p2 == m2, iota, NUM_EXPERTS), axis=1, keepdims=True)
    is2 = iota == i2
    s = m1 + m2
    comb_ref[...] = jnp.where(is1, m1 / s, 0.0) + jnp.where(is2, m2 / s, 0.0)


def _ffn_body(x_ref, w1_ref, w3_ref, w2_ref, comb_ref, out_ref):
    e = pl.program_id(0)
    ib = pl.program_id(1)
    tb = pl.program_id(2)

    @pl.when((e == 0) & (ib == 0) & (tb == 0))
    def _zero():
        out_ref[...] = jnp.zeros_like(out_ref)

    xb = x_ref[...].astype(jnp.bfloat16)
    h1 = jnp.dot(xb, w1_ref[0].astype(jnp.bfloat16),
                 preferred_element_type=jnp.float32)
    h3 = jnp.dot(xb, w3_ref[0].astype(jnp.bfloat16),
                 preferred_element_type=jnp.float32)
    g = h1 * jax.nn.sigmoid(h1) * h3
    y = jnp.dot(g.astype(jnp.bfloat16), w2_ref[0].astype(jnp.bfloat16),
                preferred_element_type=jnp.float32)
    onehot = (jax.lax.broadcasted_iota(jnp.int32, (NUM_EXPERTS, 1), 0) == e
              ).astype(jnp.float32)
    col = jnp.dot(comb_ref[...], onehot, preferred_element_type=jnp.float32)
    out_ref[pl.ds(tb * TB, TB), :] += col * y


def kernel(hidden_states, gate_w, w1s, w2s, w3s):
    B, S, H = hidden_states.shape
    x = hidden_states.reshape(-1, H)

    combine = pl.pallas_call(
        _router_body,
        grid=(T_TOKENS // TB,),
        in_specs=[
            pl.BlockSpec((TB, HIDDEN), lambda t: (t, 0)),
            pl.BlockSpec((HIDDEN, NUM_EXPERTS), lambda t: (0, 0)),
        ],
        out_specs=pl.BlockSpec((TB, NUM_EXPERTS), lambda t: (t, 0)),
        out_shape=jax.ShapeDtypeStruct((T_TOKENS, NUM_EXPERTS), jnp.float32),
    )(x, gate_w)

    out = pl.pallas_call(
        _ffn_body,
        grid=(NUM_EXPERTS, INTER // IB, T_TOKENS // TB),
        in_specs=[
            pl.BlockSpec((TB, HIDDEN), lambda e, i, t: (t, 0)),
            pl.BlockSpec((1, HIDDEN, IB), lambda e, i, t: (e, 0, i)),
            pl.BlockSpec((1, HIDDEN, IB), lambda e, i, t: (e, 0, i)),
            pl.BlockSpec((1, IB, HIDDEN), lambda e, i, t: (e, i, 0)),
            pl.BlockSpec((TB, NUM_EXPERTS), lambda e, i, t: (t, 0)),
        ],
        out_specs=pl.BlockSpec((T_TOKENS, HIDDEN), lambda e, i, t: (0, 0)),
        out_shape=jax.ShapeDtypeStruct((T_TOKENS, HIDDEN), jnp.float32),
        compiler_params=pltpu.CompilerParams(
            dimension_semantics=("arbitrary", "arbitrary", "arbitrary"),
        ),
    )(x, w1s, w3s, w2s, combine)

    return out.reshape(B, S, H)


# speedup vs baseline: 1.1175x; 1.1175x over previous
import functools

import jax
import jax.numpy as jnp
from jax import lax
from jax.experimental import pallas as pl
from jax.experimental.pallas import tpu as pltpu
from jax.experimental.pallas import tpu_sc as plsc

NUM_EXPERTS = 8
TOP_K = 2
HIDDEN = 768
INTER = 2048
T_TOKENS = 2048
N_PAIRS = T_TOKENS * TOP_K

TB = 256
BM = 128
G = N_PAIRS // BM + NUM_EXPERTS
R = G * BM

SC_CORES = 2
SC_SUBCORES = 16
NW = SC_CORES * SC_SUBCORES
LANES = 16


def _router_body(x_ref, gw_ref, idx_ref, w_ref):
    x = x_ref[...]
    logits = jnp.dot(x, gw_ref[...], preferred_element_type=jnp.float32)
    m = jnp.max(logits, axis=1, keepdims=True)
    ex = jnp.exp(logits - m)
    p = ex / jnp.sum(ex, axis=1, keepdims=True)
    iota = jax.lax.broadcasted_iota(jnp.int32, (p.shape[0], NUM_EXPERTS), 1)
    m1 = jnp.max(p, axis=1, keepdims=True)
    i1 = jnp.min(jnp.where(p == m1, iota, NUM_EXPERTS), axis=1, keepdims=True)
    is1 = iota == i1
    p2 = jnp.where(is1, -1.0, p)
    m2 = jnp.max(p2, axis=1, keepdims=True)
    i2 = jnp.min(jnp.where(p2 == m2, iota, NUM_EXPERTS), axis=1, keepdims=True)
    s = m1 + m2
    idx_ref[...] = jnp.concatenate([i1, i2], axis=1)
    w_ref[...] = jnp.concatenate([m1 / s, m2 / s], axis=1)


def _grouped_body(be_ref, xs_ref, w1_ref, w3_ref, w2_ref, rw_ref, ys_ref):
    g = pl.program_id(0)

    @pl.when(be_ref[g] < NUM_EXPERTS)
    def _():
        xb = xs_ref[...].astype(jnp.bfloat16)
        h1 = jnp.dot(xb, w1_ref[0], preferred_element_type=jnp.float32)
        h3 = jnp.dot(xb, w3_ref[0], preferred_element_type=jnp.float32)
        hh = h1 * jax.nn.sigmoid(h1) * h3
        y = jnp.dot(hh.astype(jnp.bfloat16), w2_ref[0],
                    preferred_element_type=jnp.float32)
        ys_ref[...] = y * rw_ref[...]


def _sc_gather_body(table_hbm, idx_hbm, out_hbm, idx_v, rows_v, sem):
    wid = lax.axis_index("s") * SC_CORES + lax.axis_index("c")
    bw = R // NW
    base = wid * bw
    pltpu.sync_copy(idx_hbm.at[pl.ds(base, bw)], idx_v)
    pltpu.async_copy(table_hbm.at[idx_v], rows_v, sem).wait()
    pltpu.sync_copy(rows_v, out_hbm.at[pl.ds(base, bw)])


def _sc_combine_body(ys_hbm, pos_hbm, out_hbm, idx_v, rows_v, out_v, sem):
    wid = lax.axis_index("s") * SC_CORES + lax.axis_index("c")
    tpw = T_TOKENS // NW
    chunk = 32

    for c in range(tpw // chunk):
        tok0 = wid * tpw + c * chunk
        pltpu.sync_copy(pos_hbm.at[pl.ds(tok0 * TOP_K, chunk * TOP_K)], idx_v)
        pltpu.async_copy(ys_hbm.at[idx_v], rows_v, sem).wait()

        def tok_body(i, carry):
            for j in range(HIDDEN // LANES):
                a = rows_v[2 * i, pl.ds(j * LANES, LANES)]
                b = rows_v[2 * i + 1, pl.ds(j * LANES, LANES)]
                out_v[i, pl.ds(j * LANES, LANES)] = a + b
            return carry

        lax.fori_loop(0, chunk, tok_body, 0)
        pltpu.sync_copy(out_v, out_hbm.at[pl.ds(tok0, chunk)])


def kernel(hidden_states, gate_w, w1s, w2s, w3s):
    B, S, H = hidden_states.shape
    x = hidden_states.reshape(-1, H)

    topi, topw = pl.pallas_call(
        _router_body,
        grid=(T_TOKENS // TB,),
        in_specs=[
            pl.BlockSpec((TB, HIDDEN), lambda t: (t, 0)),
            pl.BlockSpec((HIDDEN, NUM_EXPERTS), lambda t: (0, 0)),
        ],
        out_specs=[
            pl.BlockSpec((TB, TOP_K), lambda t: (t, 0)),
            pl.BlockSpec((TB, TOP_K), lambda t: (t, 0)),
        ],
        out_shape=[
            jax.ShapeDtypeStruct((T_TOKENS, TOP_K), jnp.int32),
            jax.ShapeDtypeStruct((T_TOKENS, TOP_K), jnp.float32),
        ],
    )(x, gate_w)

    ef = topi.reshape(-1)
    wf = topw.reshape(-1)
    onehot = (ef[:, None] == jnp.arange(NUM_EXPERTS)[None, :]).astype(jnp.int32)
    csum = jnp.cumsum(onehot, axis=0)
    counts = csum[-1]
    pos_in_e = jnp.take_along_axis(csum, ef[:, None], axis=1)[:, 0] - 1
    nb = (counts + BM - 1) // BM
    bstart = jnp.concatenate([jnp.zeros((1,), nb.dtype), jnp.cumsum(nb)[:-1]])
    pad_start = (bstart * BM).astype(jnp.int32)
    pos = pad_start[ef] + pos_in_e
    tok_of_pair = (jnp.arange(N_PAIRS, dtype=jnp.int32) // TOP_K)
    row_token = jnp.zeros((R,), jnp.int32).at[pos].set(tok_of_pair)
    row_w = jnp.zeros((R,), jnp.float32).at[pos].set(wf)
    nb_total = bstart[-1] + nb[-1]
    gids = jnp.arange(G, dtype=jnp.int32)
    be = (jnp.sum(gids[:, None] >= bstart[None, :], axis=1) - 1).astype(jnp.int32)
    block_expert = jnp.where(gids < nb_total, be, NUM_EXPERTS)

    sc_gather = pl.kernel(
        _sc_gather_body,
        out_type=jax.ShapeDtypeStruct((R, HIDDEN), jnp.float32),
        mesh=plsc.VectorSubcoreMesh(core_axis_name="c", subcore_axis_name="s",
                               num_cores=SC_CORES, num_subcores=SC_SUBCORES),
        scratch_types=[
            pltpu.VMEM((R // NW,), jnp.int32),
            pltpu.VMEM((R // NW, HIDDEN), jnp.float32),
            pltpu.SemaphoreType.DMA,
        ],
    )
    xs = sc_gather(x, row_token)

    w1b = w1s.astype(jnp.bfloat16)
    w2b = w2s.astype(jnp.bfloat16)
    w3b = w3s.astype(jnp.bfloat16)
    grid_spec = pltpu.PrefetchScalarGridSpec(
        num_scalar_prefetch=1,
        grid=(G,),
        in_specs=[
            pl.BlockSpec((BM, HIDDEN), lambda g, be: (g, 0)),
            pl.BlockSpec((1, HIDDEN, INTER),
                         lambda g, be: (jnp.minimum(be[g], NUM_EXPERTS - 1), 0, 0)),
            pl.BlockSpec((1, HIDDEN, INTER),
                         lambda g, be: (jnp.minimum(be[g], NUM_EXPERTS - 1), 0, 0)),
            pl.BlockSpec((1, INTER, HIDDEN),
                         lambda g, be: (jnp.minimum(be[g], NUM_EXPERTS - 1), 0, 0)),
            pl.BlockSpec((BM, 1), lambda g, be: (g, 0)),
        ],
        out_specs=pl.BlockSpec((BM, HIDDEN), lambda g, be: (g, 0)),
    )
    ys = pl.pallas_call(
        _grouped_body,
        grid_spec=grid_spec,
        out_shape=jax.ShapeDtypeStruct((R, HIDDEN), jnp.float32),
        compiler_params=pltpu.CompilerParams(
            dimension_semantics=("arbitrary",),
        ),
    )(block_expert, xs, w1b, w3b, w2b, row_w.reshape(R, 1))

    sc_combine = pl.kernel(
        _sc_combine_body,
        out_type=jax.ShapeDtypeStruct((T_TOKENS, HIDDEN), jnp.float32),
        mesh=plsc.VectorSubcoreMesh(core_axis_name="c", subcore_axis_name="s",
                               num_cores=SC_CORES, num_subcores=SC_SUBCORES),
        scratch_types=[
            pltpu.VMEM((64,), jnp.int32),
            pltpu.VMEM((64, HIDDEN), jnp.float32),
            pltpu.VMEM((32, HIDDEN), jnp.float32),
            pltpu.SemaphoreType.DMA,
        ],
    )
    out = sc_combine(ys, pos.astype(jnp.int32))

    return out.reshape(B, S, H)


# speedup vs baseline: 1.4557x; 1.3027x over previous
import functools

import jax
import jax.numpy as jnp
from jax import lax
from jax.experimental import pallas as pl
from jax.experimental.pallas import tpu as pltpu
from jax.experimental.pallas import tpu_sc as plsc

NUM_EXPERTS = 8
TOP_K = 2
HIDDEN = 768
INTER = 2048
T_TOKENS = 2048
N_PAIRS = T_TOKENS * TOP_K

TB = 256
BM = 128
G = N_PAIRS // BM + NUM_EXPERTS
R = G * BM

SC_CORES = 2
SC_SUBCORES = 16
NW = SC_CORES * SC_SUBCORES
LANES = 16


def _router_body(x_ref, gw_ref, idx_ref, w_ref):
    x = x_ref[...]
    logits = jnp.dot(x, gw_ref[...], preferred_element_type=jnp.float32)
    m = jnp.max(logits, axis=1, keepdims=True)
    ex = jnp.exp(logits - m)
    p = ex / jnp.sum(ex, axis=1, keepdims=True)
    iota = jax.lax.broadcasted_iota(jnp.int32, (p.shape[0], NUM_EXPERTS), 1)
    m1 = jnp.max(p, axis=1, keepdims=True)
    i1 = jnp.min(jnp.where(p == m1, iota, NUM_EXPERTS), axis=1, keepdims=True)
    is1 = iota == i1
    p2 = jnp.where(is1, -1.0, p)
    m2 = jnp.max(p2, axis=1, keepdims=True)
    i2 = jnp.min(jnp.where(p2 == m2, iota, NUM_EXPERTS), axis=1, keepdims=True)
    s = m1 + m2
    idx_ref[...] = jnp.concatenate([i1, i2], axis=1)
    w_ref[...] = jnp.concatenate([m1 / s, m2 / s], axis=1)


def _grouped_body(be_ref, xs_ref, w1_ref, w3_ref, w2_ref, rw_ref, ys_ref):
    g = pl.program_id(0)

    @pl.when(be_ref[g] < NUM_EXPERTS)
    def _():
        xb = xs_ref[...]
        h1 = jnp.dot(xb, w1_ref[0], preferred_element_type=jnp.float32)
        h3 = jnp.dot(xb, w3_ref[0], preferred_element_type=jnp.float32)
        hh = h1 * jax.nn.sigmoid(h1) * h3
        y = jnp.dot(hh, w2_ref[0], preferred_element_type=jnp.float32)
        ys_ref[...] = y * rw_ref[...]


GCH = 4
GCR = R // NW // GCH


def _sc_gather_body(table_hbm, idx_hbm, out_hbm, idx_v, rows0, rows1,
                    gsem0, gsem1, wsem0, wsem1):
    wid = lax.axis_index("s") * SC_CORES + lax.axis_index("c")
    bw = R // NW
    base = wid * bw
    pltpu.sync_copy(idx_hbm.at[pl.ds(wid * GCH, GCH)], idx_v)
    bufs = (rows0, rows1)
    gsems = (gsem0, gsem1)
    wsems = (wsem0, wsem1)
    del gsems, wsems
    for c in range(GCH):
        s = c % 2
        pltpu.async_copy(table_hbm.at[idx_v.at[c]], bufs[s], gsem0).wait()
        pltpu.sync_copy(bufs[s], out_hbm.at[pl.ds(base + c * GCR, GCR)])


def _sc_combine_body(ys_hbm, pos_hbm, out_hbm, idx_v, rows_v, out_v, sem):
    wid = lax.axis_index("s") * SC_CORES + lax.axis_index("c")
    tpw = T_TOKENS // NW
    chunk = 32

    for c in range(tpw // chunk):
        tok0 = wid * tpw + c * chunk
        pltpu.sync_copy(pos_hbm.at[pl.ds(tok0 * TOP_K, chunk * TOP_K)], idx_v)
        pltpu.async_copy(ys_hbm.at[idx_v], rows_v, sem).wait()

        def tok_body(i, carry):
            for j in range(HIDDEN // LANES):
                a = rows_v[2 * i, pl.ds(j * LANES, LANES)]
                b = rows_v[2 * i + 1, pl.ds(j * LANES, LANES)]
                out_v[i, pl.ds(j * LANES, LANES)] = a + b
            return carry

        lax.fori_loop(0, chunk, tok_body, 0)
        pltpu.sync_copy(out_v, out_hbm.at[pl.ds(tok0, chunk)])


def kernel(hidden_states, gate_w, w1s, w2s, w3s):
    B, S, H = hidden_states.shape
    x = hidden_states.reshape(-1, H)

    topi, topw = pl.pallas_call(
        _router_body,
        grid=(T_TOKENS // TB,),
        in_specs=[
            pl.BlockSpec((TB, HIDDEN), lambda t: (t, 0)),
            pl.BlockSpec((HIDDEN, NUM_EXPERTS), lambda t: (0, 0)),
        ],
        out_specs=[
            pl.BlockSpec((TB, TOP_K), lambda t: (t, 0)),
            pl.BlockSpec((TB, TOP_K), lambda t: (t, 0)),
        ],
        out_shape=[
            jax.ShapeDtypeStruct((T_TOKENS, TOP_K), jnp.int32),
            jax.ShapeDtypeStruct((T_TOKENS, TOP_K), jnp.float32),
        ],
    )(x, gate_w)

    ef = topi.reshape(-1)
    wf = topw.reshape(-1)
    onehot = (ef[:, None] == jnp.arange(NUM_EXPERTS)[None, :]).astype(jnp.int32)
    csum = jnp.cumsum(onehot, axis=0)
    counts = csum[-1]
    pos_in_e = jnp.sum(onehot * csum, axis=1) - 1
    nb = (counts + BM - 1) // BM
    bstart = jnp.concatenate([jnp.zeros((1,), nb.dtype), jnp.cumsum(nb)[:-1]])
    pad_start = (bstart * BM).astype(jnp.int32)
    pos = jnp.sum(onehot * pad_start[None, :], axis=1) + pos_in_e
    tok_of_pair = (jnp.arange(N_PAIRS, dtype=jnp.int32) // TOP_K)
    row_token = jnp.zeros((R,), jnp.int32).at[pos].set(
        tok_of_pair, unique_indices=True)
    row_w = jnp.zeros((R,), jnp.float32).at[pos].set(wf, unique_indices=True)
    nb_total = bstart[-1] + nb[-1]
    gids = jnp.arange(G, dtype=jnp.int32)
    be = (jnp.sum(gids[:, None] >= bstart[None, :], axis=1) - 1).astype(jnp.int32)
    block_expert = jnp.where(gids < nb_total, be, NUM_EXPERTS)

    sc_gather = pl.kernel(
        _sc_gather_body,
        out_type=jax.ShapeDtypeStruct((R, HIDDEN), jnp.float32),
        mesh=plsc.VectorSubcoreMesh(core_axis_name="c", subcore_axis_name="s",
                               num_cores=SC_CORES, num_subcores=SC_SUBCORES),
        scratch_types=[
            pltpu.VMEM((GCH, GCR), jnp.int32),
            pltpu.VMEM((GCR, HIDDEN), jnp.float32),
            pltpu.VMEM((GCR, HIDDEN), jnp.float32),
            pltpu.SemaphoreType.DMA,
            pltpu.SemaphoreType.DMA,
            pltpu.SemaphoreType.DMA,
            pltpu.SemaphoreType.DMA,
        ],
    )
    xs = sc_gather(x, row_token.reshape(R // GCR, GCR))

    grid_spec = pltpu.PrefetchScalarGridSpec(
        num_scalar_prefetch=1,
        grid=(G,),
        in_specs=[
            pl.BlockSpec((BM, HIDDEN), lambda g, be: (g, 0)),
            pl.BlockSpec((1, HIDDEN, INTER),
                         lambda g, be: (jnp.minimum(be[g], NUM_EXPERTS - 1), 0, 0)),
            pl.BlockSpec((1, HIDDEN, INTER),
                         lambda g, be: (jnp.minimum(be[g], NUM_EXPERTS - 1), 0, 0)),
            pl.BlockSpec((1, INTER, HIDDEN),
                         lambda g, be: (jnp.minimum(be[g], NUM_EXPERTS - 1), 0, 0)),
            pl.BlockSpec((BM, 1), lambda g, be: (g, 0)),
        ],
        out_specs=pl.BlockSpec((BM, HIDDEN), lambda g, be: (g, 0)),
    )
    ys = pl.pallas_call(
        _grouped_body,
        grid_spec=grid_spec,
        out_shape=jax.ShapeDtypeStruct((R, HIDDEN), jnp.float32),
        compiler_params=pltpu.CompilerParams(
            dimension_semantics=("arbitrary",),
        ),
    )(block_expert, xs, w1s, w3s, w2s, row_w.reshape(R, 1))

    sc_combine = pl.kernel(
        _sc_combine_body,
        out_type=jax.ShapeDtypeStruct((T_TOKENS, HIDDEN), jnp.float32),
        mesh=plsc.VectorSubcoreMesh(core_axis_name="c", subcore_axis_name="s",
                               num_cores=SC_CORES, num_subcores=SC_SUBCORES),
        scratch_types=[
            pltpu.VMEM((64,), jnp.int32),
            pltpu.VMEM((64, HIDDEN), jnp.float32),
            pltpu.VMEM((32, HIDDEN), jnp.float32),
            pltpu.SemaphoreType.DMA,
        ],
    )
    out = sc_combine(ys, pos.astype(jnp.int32))

    return out.reshape(B, S, H)
